# Initial kernel scaffold; baseline (speedup 1.0000x reference)
import jax
import jax.numpy as jnp
from jax.experimental import pallas as pl


def kernel(x, edge_index, ptr, W, W1, Wg, W2):
    raise NotImplementedError("write your pallas kernel here")


# speedup vs baseline: 4.2540x; 4.2540x over previous
import functools

import jax
import jax.numpy as jnp
from jax import lax
from jax.experimental import pallas as pl
from jax.experimental.pallas import tpu as pltpu
from jax.experimental.pallas import tpu_sc as plsc

N = 10000
E = 160000
DIN = 256
DOUT = 256
M = 512
G = 8
SEG = N // G
VN_ROW = SEG - 1

NC, NS, L = 2, 16, 16
CHUNK = 128
CH_PER_TILE = 80
CH_TOTAL = NS * CH_PER_TILE
E_PAD = CH_TOTAL * CHUNK
HALF = N // NC
ACC_ROWS = HALF + 8
DEG_ROWS = N + 8
STRIPE = 312
DSTRIPE = 624

_sc_mesh = plsc.VectorSubcoreMesh(core_axis_name="c", subcore_axis_name="s")


DW = 128


@functools.partial(
    pl.kernel,
    out_type=jax.ShapeDtypeStruct((N, DW), jnp.float32),
    mesh=_sc_mesh,
    scratch_types=[
        pltpu.VMEM((CH_PER_TILE, CHUNK), jnp.int32),
        pltpu.VMEM((CHUNK, DW), jnp.float32),
        pltpu.VMEM_SHARED((HALF + 8, DW), jnp.float32),
    ],
)
def _deg_kernel(dst_hbm, ones_hbm, zeros_hbm, deg_hbm, dst_v, ones_v, deg_sh):
    c = lax.axis_index("c")
    s = lax.axis_index("s")
    pltpu.sync_copy(zeros_hbm.at[pl.ds(0, STRIPE)],
                    deg_sh.at[pl.ds(s * STRIPE, STRIPE)])

    @pl.when(s == 0)
    def _():
        tail = HALF + 8 - NS * STRIPE
        pltpu.sync_copy(zeros_hbm.at[pl.ds(0, tail)],
                        deg_sh.at[pl.ds(NS * STRIPE, tail)])

    pltpu.sync_copy(ones_hbm, ones_v)
    pltpu.sync_copy(dst_hbm.at[pl.ds(s * CH_PER_TILE, CH_PER_TILE)], dst_v)

    nbase = c * HALF

    def xform(j, carry):
        for k in range(CHUNK // L):
            dd = dst_v[j, pl.ds(k * L, L)]
            local = dd - nbase
            ok = (local >= 0) & (local < HALF)
            dst_v[j, pl.ds(k * L, L)] = jnp.where(ok, local, HALF)
        return carry

    lax.fori_loop(0, CH_PER_TILE, xform, 0)
    plsc.subcore_barrier()
    for j in range(CH_PER_TILE):
        pltpu.sync_copy(ones_v, deg_sh.at[dst_v.at[j]], add=True)
    plsc.subcore_barrier()
    base = c * HALF
    pltpu.sync_copy(deg_sh.at[pl.ds(s * STRIPE, STRIPE)],
                    deg_hbm.at[pl.ds(base + s * STRIPE, STRIPE)])

    @pl.when(s == 0)
    def _():
        tail = HALF - NS * STRIPE
        pltpu.sync_copy(deg_sh.at[pl.ds(NS * STRIPE, tail)],
                        deg_hbm.at[pl.ds(base + NS * STRIPE, tail)])


DH = DOUT // 2


@functools.partial(
    pl.kernel,
    out_type=jax.ShapeDtypeStruct((N, DH), jnp.float32),
    mesh=_sc_mesh,
    scratch_types=[
        pltpu.VMEM((CH_PER_TILE, CHUNK), jnp.int32),
        pltpu.VMEM((CH_PER_TILE, CHUNK), jnp.int32),
        pltpu.VMEM((2, CHUNK, DH), jnp.float32),
        pltpu.VMEM_SHARED((ACC_ROWS, DH), jnp.float32),
        pltpu.SemaphoreType.DMA,
        pltpu.SemaphoreType.DMA,
    ],
)
def _agg_kernel(src_hbm, dst_hbm, y_hbm, zeros_hbm, agg_hbm,
                src_v, dst_v, rows_v, acc_sh, sem0, sem1):
    c = lax.axis_index("c")
    s = lax.axis_index("s")
    pltpu.sync_copy(zeros_hbm.at[pl.ds(0, STRIPE)],
                    acc_sh.at[pl.ds(s * STRIPE, STRIPE)])

    @pl.when(s == 0)
    def _():
        tail = ACC_ROWS - NS * STRIPE
        pltpu.sync_copy(zeros_hbm.at[pl.ds(0, tail)],
                        acc_sh.at[pl.ds(NS * STRIPE, tail)])

    pltpu.sync_copy(src_hbm.at[pl.ds(s * CH_PER_TILE, CH_PER_TILE)], src_v)
    pltpu.sync_copy(dst_hbm.at[pl.ds(s * CH_PER_TILE, CH_PER_TILE)], dst_v)

    nbase = c * HALF

    def xform(j, carry):
        for k in range(CHUNK // L):
            d = dst_v[j, pl.ds(k * L, L)]
            local = d - nbase
            ok = (local >= 0) & (local < HALF)
            dst_v[j, pl.ds(k * L, L)] = jnp.where(ok, local, HALF)
        return carry

    lax.fori_loop(0, CH_PER_TILE, xform, 0)
    plsc.subcore_barrier()

    sems = (sem0, sem1)
    cps = [None, None]
    cps[0] = pltpu.async_copy(y_hbm.at[src_v.at[0]], rows_v.at[0], sems[0])
    for j in range(CH_PER_TILE):
        nxt = j + 1
        if nxt < CH_PER_TILE:
            cps[nxt % 2] = pltpu.async_copy(
                y_hbm.at[src_v.at[nxt]], rows_v.at[nxt % 2], sems[nxt % 2])
        cps[j % 2].wait()
        pltpu.sync_copy(rows_v.at[j % 2], acc_sh.at[dst_v.at[j]], add=True)

    plsc.subcore_barrier()
    base = c * HALF
    pltpu.sync_copy(acc_sh.at[pl.ds(s * STRIPE, STRIPE)],
                    agg_hbm.at[pl.ds(base + s * STRIPE, STRIPE)])

    @pl.when(s == 0)
    def _():
        tail = HALF - NS * STRIPE
        pltpu.sync_copy(acc_sh.at[pl.ds(NS * STRIPE, tail)],
                        agg_hbm.at[pl.ds(base + NS * STRIPE, tail)])


def _mm_scale_body(x_ref, w_ref, deg_ref, y0_ref, y1_ref):
    xw = jnp.dot(x_ref[...], w_ref[...], preferred_element_type=jnp.float32)
    deg = deg_ref[...][:, 0:1] + 1.0
    y = xw * lax.rsqrt(deg)
    y0_ref[...] = y[:, :DH]
    y1_ref[...] = y[:, DH:]


_mm_scale = pl.pallas_call(
    _mm_scale_body,
    grid=(10,),
    in_specs=[
        pl.BlockSpec((1000, DIN), lambda i: (i, 0)),
        pl.BlockSpec((DIN, DOUT), lambda i: (0, 0)),
        pl.BlockSpec((1000, DW), lambda i: (i, 0)),
    ],
    out_specs=[
        pl.BlockSpec((1000, DH), lambda i: (i, 0)),
        pl.BlockSpec((1000, DH), lambda i: (i, 0)),
    ],
    out_shape=[
        jax.ShapeDtypeStruct((N, DH), jnp.float32),
        jax.ShapeDtypeStruct((N, DH), jnp.float32),
    ],
)


def _final_body(y0_ref, y1_ref, agg0_ref, agg1_ref, deg_ref,
                w1_ref, wg_ref, w2_ref, out_ref):
    deg = deg_ref[0][:, 0:1] + 1.0
    dinv = lax.rsqrt(deg)
    h = dinv * jnp.concatenate(
        [agg0_ref[0] + y0_ref[0], agg1_ref[0] + y1_ref[0]], axis=1)
    t = jnp.tanh(jnp.dot(h, w1_ref[...], preferred_element_type=jnp.float32))
    g = jax.nn.sigmoid(jnp.dot(h, wg_ref[...], preferred_element_type=jnp.float32))
    a = jnp.sum(t * g * w2_ref[...], axis=1, keepdims=True)
    m = jnp.max(a)
    e = jnp.exp(a - m)
    att = e / jnp.sum(e)
    ys = jnp.sum(h * att, axis=0, keepdims=True)
    rowid = lax.broadcasted_iota(jnp.int32, (SEG, 1), 0)
    out_ref[...] = jnp.where(rowid == VN_ROW, ys, h)[None]


_final = pl.pallas_call(
    _final_body,
    grid=(G,),
    in_specs=[
        pl.BlockSpec((1, SEG, DH), lambda i: (i, 0, 0)),
        pl.BlockSpec((1, SEG, DH), lambda i: (i, 0, 0)),
        pl.BlockSpec((1, SEG, DH), lambda i: (i, 0, 0)),
        pl.BlockSpec((1, SEG, DH), lambda i: (i, 0, 0)),
        pl.BlockSpec((1, SEG, DW), lambda i: (i, 0, 0)),
        pl.BlockSpec((DOUT, M), lambda i: (0, 0)),
        pl.BlockSpec((DOUT, M), lambda i: (0, 0)),
        pl.BlockSpec((1, M), lambda i: (0, 0)),
    ],
    out_specs=pl.BlockSpec((1, SEG, DOUT), lambda i: (i, 0, 0)),
    out_shape=jax.ShapeDtypeStruct((G, SEG, DOUT), jnp.float32),
)


def kernel(x, edge_index, ptr, W, W1, Wg, W2):
    src = edge_index[0]
    dst = edge_index[1]
    pad = E_PAD - E
    src_p = jnp.concatenate(
        [src, jnp.zeros((pad,), jnp.int32)]).reshape(CH_TOTAL, CHUNK)
    dst_p = jnp.concatenate(
        [dst, jnp.full((pad,), N, jnp.int32)]).reshape(CH_TOTAL, CHUNK)
    ones_in = jnp.ones((CHUNK, DW), jnp.float32)
    zeros_agg = jnp.zeros((STRIPE + 16, DH), jnp.float32)

    deg16 = _deg_kernel(dst_p, ones_in, zeros_agg)
    y0, y1 = _mm_scale(x, W, deg16)
    agg0 = _agg_kernel(src_p, dst_p, y0, zeros_agg)
    agg1 = _agg_kernel(src_p, dst_p, y1, zeros_agg)

    out = _final(
        y0.reshape(G, SEG, DH),
        y1.reshape(G, SEG, DH),
        agg0.reshape(G, SEG, DH),
        agg1.reshape(G, SEG, DH),
        deg16.reshape(G, SEG, DW),
        W1, Wg, W2.reshape(1, M),
    )
    return out.reshape(N, DOUT)


# speedup vs baseline: 14.9074x; 3.5043x over previous
import functools

import jax
import jax.numpy as jnp
from jax import lax
from jax.experimental import pallas as pl
from jax.experimental.pallas import tpu as pltpu
from jax.experimental.pallas import tpu_sc as plsc

N = 10000
E = 160000
DIN = 256
DOUT = 256
M = 512
G = 8
SEG = N // G
VN_ROW = SEG - 1

NC, NS, L = 2, 16, 16
CHUNK = 128
CH_PER_TILE = 80
CH_TOTAL = NS * CH_PER_TILE
E_PAD = CH_TOTAL * CHUNK
HALF = N // NC
ACC_ROWS = HALF + 8
DEG_ROWS = N + 8
STRIPE = 312
IGN = -1
NSLOT = 4
AHEAD = 2

_sc_mesh = plsc.VectorSubcoreMesh(core_axis_name="c", subcore_axis_name="s")


DW = 128


@functools.partial(
    pl.kernel,
    out_type=jax.ShapeDtypeStruct((N, DW), jnp.float32),
    mesh=_sc_mesh,
    scratch_types=[
        pltpu.VMEM((CH_PER_TILE, CHUNK), jnp.int32),
        pltpu.VMEM((CHUNK, DW), jnp.float32),
        pltpu.VMEM_SHARED((HALF + 8, DW), jnp.float32),
    ],
)
def _deg_kernel(dst_hbm, ones_hbm, zeros_hbm, deg_hbm, dst_v, ones_v, deg_sh):
    c = lax.axis_index("c")
    s = lax.axis_index("s")
    pltpu.sync_copy(zeros_hbm.at[pl.ds(0, STRIPE)],
                    deg_sh.at[pl.ds(s * STRIPE, STRIPE)])

    @pl.when(s == 0)
    def _():
        tail = HALF + 8 - NS * STRIPE
        pltpu.sync_copy(zeros_hbm.at[pl.ds(0, tail)],
                        deg_sh.at[pl.ds(NS * STRIPE, tail)])

    pltpu.sync_copy(ones_hbm, ones_v)
    pltpu.sync_copy(dst_hbm.at[pl.ds(s * CH_PER_TILE, CH_PER_TILE)], dst_v)

    nbase = c * HALF

    def xform(j, carry):
        for k in range(CHUNK // L):
            dd = dst_v[j, pl.ds(k * L, L)]
            local = dd - nbase
            ok = (local >= 0) & (local < HALF)
            dst_v[j, pl.ds(k * L, L)] = jnp.where(ok, local, IGN)
        return carry

    lax.fori_loop(0, CH_PER_TILE, xform, 0)
    plsc.subcore_barrier()
    for j in range(CH_PER_TILE):
        pltpu.sync_copy(ones_v,
                        deg_sh.at[plsc.Indices(dst_v.at[j], ignored_value=IGN)],
                        add=True)
    plsc.subcore_barrier()
    base = c * HALF
    pltpu.sync_copy(deg_sh.at[pl.ds(s * STRIPE, STRIPE)],
                    deg_hbm.at[pl.ds(base + s * STRIPE, STRIPE)])

    @pl.when(s == 0)
    def _():
        tail = HALF - NS * STRIPE
        pltpu.sync_copy(deg_sh.at[pl.ds(NS * STRIPE, tail)],
                        deg_hbm.at[pl.ds(base + NS * STRIPE, tail)])


DH = DOUT // 2


@functools.partial(
    pl.kernel,
    out_type=jax.ShapeDtypeStruct((N, DH), jnp.float32),
    mesh=_sc_mesh,
    scratch_types=[
        pltpu.VMEM((CH_PER_TILE, CHUNK), jnp.int32),
        pltpu.VMEM((CH_PER_TILE, CHUNK), jnp.int32),
        pltpu.VMEM((NSLOT, CHUNK, DH), jnp.float32),
        pltpu.VMEM_SHARED((ACC_ROWS, DH), jnp.float32),
        [pltpu.SemaphoreType.DMA] * NSLOT,
        [pltpu.SemaphoreType.DMA] * NSLOT,
    ],
)
def _agg_kernel(src_hbm, dst_hbm, y_hbm, zeros_hbm, agg_hbm,
                src_v, dst_v, rows_v, acc_sh, gsems, ssems):
    c = lax.axis_index("c")
    s = lax.axis_index("s")
    pltpu.sync_copy(zeros_hbm.at[pl.ds(0, STRIPE)],
                    acc_sh.at[pl.ds(s * STRIPE, STRIPE)])

    @pl.when(s == 0)
    def _():
        tail = ACC_ROWS - NS * STRIPE
        pltpu.sync_copy(zeros_hbm.at[pl.ds(0, tail)],
                        acc_sh.at[pl.ds(NS * STRIPE, tail)])

    pltpu.sync_copy(src_hbm.at[pl.ds(s * CH_PER_TILE, CH_PER_TILE)], src_v)
    pltpu.sync_copy(dst_hbm.at[pl.ds(s * CH_PER_TILE, CH_PER_TILE)], dst_v)

    nbase = c * HALF

    def xform(j, carry):
        for k in range(CHUNK // L):
            d = dst_v[j, pl.ds(k * L, L)]
            sv = src_v[j, pl.ds(k * L, L)]
            local = d - nbase
            ok = (local >= 0) & (local < HALF)
            dst_v[j, pl.ds(k * L, L)] = jnp.where(ok, local, IGN)
            src_v[j, pl.ds(k * L, L)] = jnp.where(ok, sv, IGN)
        return carry

    lax.fori_loop(0, CH_PER_TILE, xform, 0)
    plsc.subcore_barrier()

    def start_gather(j):
        return pltpu.async_copy(
            y_hbm.at[plsc.Indices(src_v.at[j], ignored_value=IGN)],
            rows_v.at[j % NSLOT], gsems[j % NSLOT])

    def start_scatter(j):
        return pltpu.async_copy(
            rows_v.at[j % NSLOT],
            acc_sh.at[plsc.Indices(dst_v.at[j], ignored_value=IGN)],
            ssems[j % NSLOT], add=True)

    gathers = [None] * CH_PER_TILE
    scatters = [None] * CH_PER_TILE
    for j in range(min(AHEAD, CH_PER_TILE)):
        gathers[j] = start_gather(j)
    for j in range(CH_PER_TILE):
        nxt = j + AHEAD
        if nxt < CH_PER_TILE:
            if nxt >= NSLOT:
                scatters[nxt - NSLOT].wait()
            gathers[nxt] = start_gather(nxt)
        gathers[j].wait()
        scatters[j] = start_scatter(j)
    for j in range(max(0, CH_PER_TILE - NSLOT), CH_PER_TILE):
        scatters[j].wait()

    plsc.subcore_barrier()
    base = c * HALF
    pltpu.sync_copy(acc_sh.at[pl.ds(s * STRIPE, STRIPE)],
                    agg_hbm.at[pl.ds(base + s * STRIPE, STRIPE)])

    @pl.when(s == 0)
    def _():
        tail = HALF - NS * STRIPE
        pltpu.sync_copy(acc_sh.at[pl.ds(NS * STRIPE, tail)],
                        agg_hbm.at[pl.ds(base + NS * STRIPE, tail)])


def _mm_scale_body(x_ref, w_ref, deg_ref, y0_ref, y1_ref):
    xw = jnp.dot(x_ref[...], w_ref[...], preferred_element_type=jnp.float32)
    deg = deg_ref[...][:, 0:1] + 1.0
    y = xw * lax.rsqrt(deg)
    y0_ref[...] = y[:, :DH]
    y1_ref[...] = y[:, DH:]


_mm_scale = pl.pallas_call(
    _mm_scale_body,
    grid=(10,),
    in_specs=[
        pl.BlockSpec((1000, DIN), lambda i: (i, 0)),
        pl.BlockSpec((DIN, DOUT), lambda i: (0, 0)),
        pl.BlockSpec((1000, DW), lambda i: (i, 0)),
    ],
    out_specs=[
        pl.BlockSpec((1000, DH), lambda i: (i, 0)),
        pl.BlockSpec((1000, DH), lambda i: (i, 0)),
    ],
    out_shape=[
        jax.ShapeDtypeStruct((N, DH), jnp.float32),
        jax.ShapeDtypeStruct((N, DH), jnp.float32),
    ],
)


def _final_body(y0_ref, y1_ref, agg0_ref, agg1_ref, deg_ref,
                w1_ref, wg_ref, w2_ref, out_ref):
    deg = deg_ref[0][:, 0:1] + 1.0
    dinv = lax.rsqrt(deg)
    h = dinv * jnp.concatenate(
        [agg0_ref[0] + y0_ref[0], agg1_ref[0] + y1_ref[0]], axis=1)
    t = jnp.tanh(jnp.dot(h, w1_ref[...], preferred_element_type=jnp.float32))
    g = jax.nn.sigmoid(jnp.dot(h, wg_ref[...], preferred_element_type=jnp.float32))
    a = jnp.sum(t * g * w2_ref[...], axis=1, keepdims=True)
    m = jnp.max(a)
    e = jnp.exp(a - m)
    att = e / jnp.sum(e)
    ys = jnp.sum(h * att, axis=0, keepdims=True)
    rowid = lax.broadcasted_iota(jnp.int32, (SEG, 1), 0)
    out_ref[...] = jnp.where(rowid == VN_ROW, ys, h)[None]


_final = pl.pallas_call(
    _final_body,
    grid=(G,),
    in_specs=[
        pl.BlockSpec((1, SEG, DH), lambda i: (i, 0, 0)),
        pl.BlockSpec((1, SEG, DH), lambda i: (i, 0, 0)),
        pl.BlockSpec((1, SEG, DH), lambda i: (i, 0, 0)),
        pl.BlockSpec((1, SEG, DH), lambda i: (i, 0, 0)),
        pl.BlockSpec((1, SEG, DW), lambda i: (i, 0, 0)),
        pl.BlockSpec((DOUT, M), lambda i: (0, 0)),
        pl.BlockSpec((DOUT, M), lambda i: (0, 0)),
        pl.BlockSpec((1, M), lambda i: (0, 0)),
    ],
    out_specs=pl.BlockSpec((1, SEG, DOUT), lambda i: (i, 0, 0)),
    out_shape=jax.ShapeDtypeStruct((G, SEG, DOUT), jnp.float32),
)


def kernel(x, edge_index, ptr, W, W1, Wg, W2):
    src = edge_index[0]
    dst = edge_index[1]
    pad = E_PAD - E
    src_p = jnp.concatenate(
        [src, jnp.zeros((pad,), jnp.int32)]).reshape(CH_TOTAL, CHUNK)
    dst_p = jnp.concatenate(
        [dst, jnp.full((pad,), N, jnp.int32)]).reshape(CH_TOTAL, CHUNK)
    ones_in = jnp.ones((CHUNK, DW), jnp.float32)
    zeros_agg = jnp.zeros((STRIPE + 16, DH), jnp.float32)

    deg16 = _deg_kernel(dst_p, ones_in, zeros_agg)
    y0, y1 = _mm_scale(x, W, deg16)
    agg0 = _agg_kernel(src_p, dst_p, y0, zeros_agg)
    agg1 = _agg_kernel(src_p, dst_p, y1, zeros_agg)

    out = _final(
        y0.reshape(G, SEG, DH),
        y1.reshape(G, SEG, DH),
        agg0.reshape(G, SEG, DH),
        agg1.reshape(G, SEG, DH),
        deg16.reshape(G, SEG, DW),
        W1, Wg, W2.reshape(1, M),
    )
    return out.reshape(N, DOUT)


# speedup vs baseline: 15.6304x; 1.0485x over previous
import functools

import jax
import jax.numpy as jnp
from jax import lax
from jax.experimental import pallas as pl
from jax.experimental.pallas import tpu as pltpu
from jax.experimental.pallas import tpu_sc as plsc

N = 10000
E = 160000
DIN = 256
DOUT = 256
M = 512
G = 8
SEG = N // G
VN_ROW = SEG - 1

NC, NS, L = 2, 16, 16
CHUNK = 128
CH_PER_TILE = 80
CH_TOTAL = NS * CH_PER_TILE
E_PAD = CH_TOTAL * CHUNK
HALF = N // NC
ACC_ROWS = HALF + 8
DEG_ROWS = N + 8
STRIPE = 312
IGN = -1
NSLOT = 4
AHEAD = 2

_sc_mesh = plsc.VectorSubcoreMesh(core_axis_name="c", subcore_axis_name="s")


DW = 128


NSEM = 8


@functools.partial(
    pl.kernel,
    out_type=jax.ShapeDtypeStruct((N, DW), jnp.float32),
    mesh=_sc_mesh,
    scratch_types=[
        pltpu.VMEM((CH_PER_TILE, CHUNK), jnp.int32),
        pltpu.VMEM((CHUNK, DW), jnp.float32),
        pltpu.VMEM_SHARED((HALF + 8, DW), jnp.float32),
        [pltpu.SemaphoreType.DMA] * NSEM,
    ],
)
def _deg_kernel(dst_hbm, ones_hbm, zeros_hbm, deg_hbm, dst_v, ones_v, deg_sh,
                sems):
    c = lax.axis_index("c")
    s = lax.axis_index("s")
    pltpu.sync_copy(zeros_hbm.at[pl.ds(0, STRIPE)],
                    deg_sh.at[pl.ds(s * STRIPE, STRIPE)])

    @pl.when(s == 0)
    def _():
        tail = HALF + 8 - NS * STRIPE
        pltpu.sync_copy(zeros_hbm.at[pl.ds(0, tail)],
                        deg_sh.at[pl.ds(NS * STRIPE, tail)])

    pltpu.sync_copy(ones_hbm, ones_v)
    pltpu.sync_copy(dst_hbm.at[pl.ds(s * CH_PER_TILE, CH_PER_TILE)], dst_v)

    nbase = c * HALF

    def xform(j, carry):
        for k in range(CHUNK // L):
            dd = dst_v[j, pl.ds(k * L, L)]
            local = dd - nbase
            ok = (local >= 0) & (local < HALF)
            dst_v[j, pl.ds(k * L, L)] = jnp.where(ok, local, IGN)
        return carry

    lax.fori_loop(0, CH_PER_TILE, xform, 0)
    plsc.subcore_barrier()
    descs = [None] * CH_PER_TILE
    for j in range(CH_PER_TILE):
        if j >= NSEM:
            descs[j - NSEM].wait()
        descs[j] = pltpu.async_copy(
            ones_v, deg_sh.at[plsc.Indices(dst_v.at[j], ignored_value=IGN)],
            sems[j % NSEM], add=True)
    for j in range(CH_PER_TILE - NSEM, CH_PER_TILE):
        descs[j].wait()
    plsc.subcore_barrier()
    base = c * HALF
    pltpu.sync_copy(deg_sh.at[pl.ds(s * STRIPE, STRIPE)],
                    deg_hbm.at[pl.ds(base + s * STRIPE, STRIPE)])

    @pl.when(s == 0)
    def _():
        tail = HALF - NS * STRIPE
        pltpu.sync_copy(deg_sh.at[pl.ds(NS * STRIPE, tail)],
                        deg_hbm.at[pl.ds(base + NS * STRIPE, tail)])


DH = DOUT // 2


@functools.partial(
    pl.kernel,
    out_type=(jax.ShapeDtypeStruct((N, DH), jnp.float32),
              jax.ShapeDtypeStruct((N, DH), jnp.float32)),
    mesh=_sc_mesh,
    scratch_types=[
        pltpu.VMEM((CH_PER_TILE, CHUNK), jnp.int32),
        pltpu.VMEM((CH_PER_TILE, CHUNK), jnp.int32),
        pltpu.VMEM((NSLOT, CHUNK, DH), jnp.float32),
        pltpu.VMEM_SHARED((ACC_ROWS, DH), jnp.float32),
        [pltpu.SemaphoreType.DMA] * NSLOT,
        [pltpu.SemaphoreType.DMA] * NSLOT,
    ],
)
def _agg_kernel(src_hbm, dst_hbm, y0_hbm, y1_hbm, zeros_hbm,
                agg0_hbm, agg1_hbm, src_v, dst_v, rows_v, acc_sh,
                gsems, ssems):
    c = lax.axis_index("c")
    s = lax.axis_index("s")

    def zero_acc():
        pltpu.sync_copy(zeros_hbm.at[pl.ds(0, STRIPE)],
                        acc_sh.at[pl.ds(s * STRIPE, STRIPE)])

        @pl.when(s == 0)
        def _():
            tail = ACC_ROWS - NS * STRIPE
            pltpu.sync_copy(zeros_hbm.at[pl.ds(0, tail)],
                            acc_sh.at[pl.ds(NS * STRIPE, tail)])

    zero_acc()
    pltpu.sync_copy(src_hbm.at[pl.ds(s * CH_PER_TILE, CH_PER_TILE)], src_v)
    pltpu.sync_copy(dst_hbm.at[pl.ds(s * CH_PER_TILE, CH_PER_TILE)], dst_v)

    nbase = c * HALF

    def xform(j, carry):
        for k in range(CHUNK // L):
            d = dst_v[j, pl.ds(k * L, L)]
            sv = src_v[j, pl.ds(k * L, L)]
            local = d - nbase
            ok = (local >= 0) & (local < HALF)
            dst_v[j, pl.ds(k * L, L)] = jnp.where(ok, local, IGN)
            src_v[j, pl.ds(k * L, L)] = jnp.where(ok, sv, IGN)
        return carry

    lax.fori_loop(0, CH_PER_TILE, xform, 0)
    plsc.subcore_barrier()

    base = c * HALF

    def run_pass(y_hbm, agg_hbm):
        def start_gather(j):
            return pltpu.async_copy(
                y_hbm.at[plsc.Indices(src_v.at[j], ignored_value=IGN)],
                rows_v.at[j % NSLOT], gsems[j % NSLOT])

        def start_scatter(j):
            return pltpu.async_copy(
                rows_v.at[j % NSLOT],
                acc_sh.at[plsc.Indices(dst_v.at[j], ignored_value=IGN)],
                ssems[j % NSLOT], add=True)

        gathers = [None] * CH_PER_TILE
        scatters = [None] * CH_PER_TILE
        for j in range(min(AHEAD, CH_PER_TILE)):
            gathers[j] = start_gather(j)
        for j in range(CH_PER_TILE):
            nxt = j + AHEAD
            if nxt < CH_PER_TILE:
                if nxt >= NSLOT:
                    scatters[nxt - NSLOT].wait()
                gathers[nxt] = start_gather(nxt)
            gathers[j].wait()
            scatters[j] = start_scatter(j)
        for j in range(max(0, CH_PER_TILE - NSLOT), CH_PER_TILE):
            scatters[j].wait()

        plsc.subcore_barrier()
        pltpu.sync_copy(acc_sh.at[pl.ds(s * STRIPE, STRIPE)],
                        agg_hbm.at[pl.ds(base + s * STRIPE, STRIPE)])

        @pl.when(s == 0)
        def _():
            tail = HALF - NS * STRIPE
            pltpu.sync_copy(acc_sh.at[pl.ds(NS * STRIPE, tail)],
                            agg_hbm.at[pl.ds(base + NS * STRIPE, tail)])

    run_pass(y0_hbm, agg0_hbm)
    zero_acc()
    plsc.subcore_barrier()
    run_pass(y1_hbm, agg1_hbm)


def _mm_scale_body(x_ref, w_ref, deg_ref, y0_ref, y1_ref):
    xw = jnp.dot(x_ref[...], w_ref[...], preferred_element_type=jnp.float32)
    deg = deg_ref[...][:, 0:1] + 1.0
    y = xw * lax.rsqrt(deg)
    y0_ref[...] = y[:, :DH]
    y1_ref[...] = y[:, DH:]


_mm_scale = pl.pallas_call(
    _mm_scale_body,
    grid=(10,),
    in_specs=[
        pl.BlockSpec((1000, DIN), lambda i: (i, 0)),
        pl.BlockSpec((DIN, DOUT), lambda i: (0, 0)),
        pl.BlockSpec((1000, DW), lambda i: (i, 0)),
    ],
    out_specs=[
        pl.BlockSpec((1000, DH), lambda i: (i, 0)),
        pl.BlockSpec((1000, DH), lambda i: (i, 0)),
    ],
    out_shape=[
        jax.ShapeDtypeStruct((N, DH), jnp.float32),
        jax.ShapeDtypeStruct((N, DH), jnp.float32),
    ],
)


def _final_body(y0_ref, y1_ref, agg0_ref, agg1_ref, deg_ref,
                w1_ref, wg_ref, w2_ref, out_ref):
    deg = deg_ref[0][:, 0:1] + 1.0
    dinv = lax.rsqrt(deg)
    h = dinv * jnp.concatenate(
        [agg0_ref[0] + y0_ref[0], agg1_ref[0] + y1_ref[0]], axis=1)
    t = jnp.tanh(jnp.dot(h, w1_ref[...], preferred_element_type=jnp.float32))
    g = jax.nn.sigmoid(jnp.dot(h, wg_ref[...], preferred_element_type=jnp.float32))
    a = jnp.sum(t * g * w2_ref[...], axis=1, keepdims=True)
    m = jnp.max(a)
    e = jnp.exp(a - m)
    att = e / jnp.sum(e)
    ys = jnp.sum(h * att, axis=0, keepdims=True)
    rowid = lax.broadcasted_iota(jnp.int32, (SEG, 1), 0)
    out_ref[...] = jnp.where(rowid == VN_ROW, ys, h)[None]


_final = pl.pallas_call(
    _final_body,
    grid=(G,),
    in_specs=[
        pl.BlockSpec((1, SEG, DH), lambda i: (i, 0, 0)),
        pl.BlockSpec((1, SEG, DH), lambda i: (i, 0, 0)),
        pl.BlockSpec((1, SEG, DH), lambda i: (i, 0, 0)),
        pl.BlockSpec((1, SEG, DH), lambda i: (i, 0, 0)),
        pl.BlockSpec((1, SEG, DW), lambda i: (i, 0, 0)),
        pl.BlockSpec((DOUT, M), lambda i: (0, 0)),
        pl.BlockSpec((DOUT, M), lambda i: (0, 0)),
        pl.BlockSpec((1, M), lambda i: (0, 0)),
    ],
    out_specs=pl.BlockSpec((1, SEG, DOUT), lambda i: (i, 0, 0)),
    out_shape=jax.ShapeDtypeStruct((G, SEG, DOUT), jnp.float32),
)


def kernel(x, edge_index, ptr, W, W1, Wg, W2):
    src = edge_index[0]
    dst = edge_index[1]
    pad = E_PAD - E
    src_p = jnp.concatenate(
        [src, jnp.zeros((pad,), jnp.int32)]).reshape(CH_TOTAL, CHUNK)
    dst_p = jnp.concatenate(
        [dst, jnp.full((pad,), N, jnp.int32)]).reshape(CH_TOTAL, CHUNK)
    ones_in = jnp.ones((CHUNK, DW), jnp.float32)
    zeros_agg = jnp.zeros((STRIPE + 16, DH), jnp.float32)

    deg1 = _deg_kernel(dst_p, ones_in, zeros_agg)
    y0, y1 = _mm_scale(x, W, deg1)
    agg0, agg1 = _agg_kernel(src_p, dst_p, y0, y1, zeros_agg)

    out = _final(
        y0.reshape(G, SEG, DH),
        y1.reshape(G, SEG, DH),
        agg0.reshape(G, SEG, DH),
        agg1.reshape(G, SEG, DH),
        deg1.reshape(G, SEG, DW),
        W1, Wg, W2.reshape(1, M),
    )
    return out.reshape(N, DOUT)
